# baseline (speedup 1.0000x reference)
import functools

import jax
import jax.numpy as jnp
from jax import lax
from jax.experimental import pallas as pl
from jax.experimental.pallas import tpu as pltpu
from jax.experimental.pallas import tpu_sc as plsc

_B = 16384
_D = 64
_NC = 2
_NS = 16
_NW = _NC * _NS
_BPW = _B // _NW
_CHUNK = 128
_NCHUNK = _BPW // _CHUNK


def _body(users_hbm, items_hbm, ut_hbm, it_hbm, out_hbm,
          uidx_v, iidx_v, urows_v, irows_v, out_v, tile_v, sem_u, sem_i):
    cid = lax.axis_index("c")
    sid = lax.axis_index("s")
    wid = sid * _NC + cid
    base = wid * _BPW

    for j in range(_NCHUNK):
        pltpu.sync_copy(users_hbm.at[pl.ds(base + j * _CHUNK, _CHUNK)],
                        uidx_v.at[j])
        pltpu.sync_copy(items_hbm.at[pl.ds(base + j * _CHUNK, _CHUNK)],
                        iidx_v.at[j])

    copies = []
    for j in range(_NCHUNK):
        copies.append(pltpu.async_copy(
            ut_hbm.at[uidx_v.at[j]],
            urows_v.at[pl.ds(j * _CHUNK, _CHUNK)], sem_u))
        copies.append(pltpu.async_copy(
            it_hbm.at[iidx_v.at[j]],
            irows_v.at[pl.ds(j * _CHUNK, _CHUNK)], sem_i))
    for cp in copies:
        cp.wait()

    lanes = lax.iota(jnp.int32, 16)

    def group_body(g, carry):
        e0 = g * 16
        for i in range(16):
            e = e0 + i
            acc = urows_v[e, 0:16] * irows_v[e, 0:16]
            for c in range(1, 4):
                acc = acc + urows_v[e, c * 16:(c + 1) * 16] * \
                    irows_v[e, c * 16:(c + 1) * 16]
            plsc.store_scatter(tile_v, [lanes, jnp.full((16,), i, jnp.int32)],
                               acc)
        rowsum = tile_v[0, :]
        for r in range(1, 16):
            rowsum = rowsum + tile_v[r, :]
        out_v[pl.ds(e0, 16)] = rowsum
        return carry

    lax.fori_loop(0, _BPW // 16, group_body, 0)

    pltpu.sync_copy(out_v, out_hbm.at[pl.ds(base, _BPW)])


@jax.jit
def _run(users, items, user_table, item_table):
    mesh = plsc.VectorSubcoreMesh(core_axis_name="c", subcore_axis_name="s")
    f = pl.kernel(
        _body,
        mesh=mesh,
        out_type=jax.ShapeDtypeStruct((_B,), jnp.float32),
        scratch_types=[
            pltpu.VMEM((_NCHUNK, _CHUNK), jnp.int32),
            pltpu.VMEM((_NCHUNK, _CHUNK), jnp.int32),
            pltpu.VMEM((_BPW, _D), jnp.float32),
            pltpu.VMEM((_BPW, _D), jnp.float32),
            pltpu.VMEM((_BPW,), jnp.float32),
            pltpu.VMEM((16, 16), jnp.float32),
            pltpu.SemaphoreType.DMA,
            pltpu.SemaphoreType.DMA,
        ],
        compiler_params=pltpu.CompilerParams(
            needs_layout_passes=False, use_tc_tiling_on_sc=False),
    )
    return f(users, items, user_table, item_table)


def kernel(users, items, user_table, item_table):
    return _run(users.astype(jnp.int32), items.astype(jnp.int32),
                user_table, item_table)


# speedup vs baseline: 2.1756x; 2.1756x over previous
import functools

import jax
import jax.numpy as jnp
from jax import lax
from jax.experimental import pallas as pl
from jax.experimental.pallas import tpu as pltpu
from jax.experimental.pallas import tpu_sc as plsc

_B = 16384
_D = 64
_NC = 2
_NS = 16
_NW = _NC * _NS
_BPW = _B // _NW
_WAVE = 32
_NWAVE = _BPW // _WAVE


def _body(users_hbm, items_hbm, ut3_hbm, it3_hbm, out_hbm,
          uidx_v, iidx_v, ublk_v, iblk_v, out_v, tile_v, sem_u, sem_i):
    cid = lax.axis_index("c")
    sid = lax.axis_index("s")
    wid = sid * _NC + cid
    base = wid * _BPW

    pltpu.sync_copy(users_hbm.at[pl.ds(base, _BPW)], uidx_v)
    pltpu.sync_copy(items_hbm.at[pl.ds(base, _BPW)], iidx_v)

    lanes = lax.iota(jnp.int32, 16)

    def wave_body(w, carry):
        e0 = w * _WAVE
        for g in range(_WAVE // 16):
            ut_vec = lax.shift_right_logical(
                uidx_v[pl.ds(e0 + g * 16, 16)], 3)
            it_vec = lax.shift_right_logical(
                iidx_v[pl.ds(e0 + g * 16, 16)], 3)
            for i in range(16):
                j = g * 16 + i
                pltpu.async_copy(ut3_hbm.at[ut_vec[i]], ublk_v.at[j], sem_u)
                pltpu.async_copy(it3_hbm.at[it_vec[i]], iblk_v.at[j], sem_i)
        pltpu.make_async_copy(ut3_hbm.at[pl.ds(0, _WAVE)], ublk_v,
                              sem_u).wait()
        pltpu.make_async_copy(it3_hbm.at[pl.ds(0, _WAVE)], iblk_v,
                              sem_i).wait()

        for g in range(_WAVE // 16):
            us_vec = jnp.bitwise_and(uidx_v[pl.ds(e0 + g * 16, 16)], 7)
            is_vec = jnp.bitwise_and(iidx_v[pl.ds(e0 + g * 16, 16)], 7)
            for i in range(16):
                j = g * 16 + i
                su = us_vec[i]
                si = is_vec[i]
                acc = ublk_v[j, su, pl.ds(0, 16)] * \
                    iblk_v[j, si, pl.ds(0, 16)]
                for c in range(1, 4):
                    acc = acc + ublk_v[j, su, pl.ds(c * 16, 16)] * \
                        iblk_v[j, si, pl.ds(c * 16, 16)]
                plsc.store_scatter(tile_v, [lanes * 16 + i], acc)
            rowsum = tile_v[pl.ds(0, 16)]
            for r in range(1, 16):
                rowsum = rowsum + tile_v[pl.ds(r * 16, 16)]
            out_v[pl.ds(e0 + g * 16, 16)] = rowsum
        return carry

    lax.fori_loop(0, _NWAVE, wave_body, 0)

    pltpu.sync_copy(out_v, out_hbm.at[pl.ds(base, _BPW)])


@jax.jit
def _run(users, items, user_table, item_table):
    mesh = plsc.VectorSubcoreMesh(core_axis_name="c", subcore_axis_name="s")
    f = pl.kernel(
        _body,
        mesh=mesh,
        out_type=jax.ShapeDtypeStruct((_B,), jnp.float32),
        scratch_types=[
            pltpu.VMEM((_BPW,), jnp.int32),
            pltpu.VMEM((_BPW,), jnp.int32),
            pltpu.VMEM((_WAVE, 8, _D), jnp.float32),
            pltpu.VMEM((_WAVE, 8, _D), jnp.float32),
            pltpu.VMEM((_BPW,), jnp.float32),
            pltpu.VMEM((256,), jnp.float32),
            pltpu.SemaphoreType.DMA,
            pltpu.SemaphoreType.DMA,
        ],
        compiler_params=pltpu.CompilerParams(needs_layout_passes=False),
    )
    ut3 = user_table.reshape(125000, 8, _D)
    it3 = item_table.reshape(125000, 8, _D)
    return f(users, items, ut3, it3)


def kernel(users, items, user_table, item_table):
    return _run(users.astype(jnp.int32), items.astype(jnp.int32),
                user_table, item_table)
